# baseline (speedup 1.0000x reference)
import functools

import jax
import jax.numpy as jnp
from jax import lax
from jax.experimental import pallas as pl
from jax.experimental.pallas import tpu as pltpu
from jax.experimental.pallas import tpu_sc as plsc

_N_IMAGES = 150000
_BATCH = 16384
_ROT_D = 36
_TRA_D = 18
_ROT_S = 64
_TRA_S = 32
_L = 16
_W = 128

_NW = 32
_B_PER_W = _BATCH // _NW
_SUB = 128
_NSUB = _B_PER_W // _SUB


def _make_gather():
    mesh = plsc.VectorSubcoreMesh(core_axis_name="c", subcore_axis_name="s")

    @functools.partial(
        pl.kernel,
        mesh=mesh,
        compiler_params=pltpu.CompilerParams(
            use_tc_tiling_on_sc=False, needs_layout_passes=False),
        out_type=[
            jax.ShapeDtypeStruct((6, 8, _BATCH), jnp.float32),
            jax.ShapeDtypeStruct((6, 4, _BATCH), jnp.float32),
        ],
        scratch_types=[
            pltpu.VMEM((_B_PER_W,), jnp.int32),
            pltpu.VMEM((_NSUB, _SUB), jnp.int32),
            pltpu.VMEM((_NSUB, _SUB), jnp.int32),
            pltpu.VMEM((_B_PER_W // _L, _L), jnp.int32),
            pltpu.VMEM((_B_PER_W // _L, _L), jnp.int32),
            pltpu.VMEM((_SUB, _W), jnp.float32),
            pltpu.VMEM((_SUB, _W), jnp.float32),
            pltpu.VMEM((_ROT_D * _B_PER_W,), jnp.float32),
            pltpu.VMEM((_TRA_D * _B_PER_W,), jnp.float32),
            pltpu.SemaphoreType.DMA,
        ],
    )
    def gather_kernel(rot_hbm, tra_hbm, idx_hbm, rot_out, tra_out,
                      idx_v, rot_gi, tra_gi, rr, rt,
                      rot_stage, tra_stage, rot_pack, tra_pack, sem):
        wid = lax.axis_index("s") * 2 + lax.axis_index("c")
        base = wid * _B_PER_W
        iota = lax.iota(jnp.int32, _L)

        pltpu.sync_copy(idx_hbm.at[pl.ds(base, _B_PER_W)], idx_v)

        for c in range(_B_PER_W // _L):
            iv = idx_v[pl.ds(c * _L, _L)]
            rot_gi[c >> 3, pl.ds((c & 7) * _L, _L)] = iv >> 1
            tra_gi[c >> 3, pl.ds((c & 7) * _L, _L)] = iv >> 2
            rr[c] = (iv & 1) << 6
            rt[c] = (iv & 3) << 5

        for b in range(_NSUB):
            cr = pltpu.async_copy(rot_hbm.at[rot_gi.at[b]], rot_stage, sem)
            ct = pltpu.async_copy(tra_hbm.at[tra_gi.at[b]], tra_stage, sem)
            cr.wait()
            ct.wait()

            def rot_body(it, _):
                g = it & 7
                s = it >> 3
                j = iota + g * _L
                src = (j << 7) + s + plsc.load_gather(
                    rr, [(j >> 4) + 8 * b, j & 15])
                vals = plsc.load_gather(rot_stage, [src >> 7, src & 127])
                rot_pack[pl.ds(s * _B_PER_W + b * _SUB + g * _L, _L)] = vals
                return 0

            lax.fori_loop(0, _ROT_D * (_SUB // _L), rot_body, 0)

            def tra_body(it, _):
                g = it & 7
                s = it >> 3
                j = iota + g * _L
                src = (j << 7) + s + plsc.load_gather(
                    rt, [(j >> 4) + 8 * b, j & 15])
                vals = plsc.load_gather(tra_stage, [src >> 7, src & 127])
                tra_pack[pl.ds(s * _B_PER_W + b * _SUB + g * _L, _L)] = vals
                return 0

            lax.fori_loop(0, _TRA_D * (_SUB // _L), tra_body, 0)

        for s in range(_ROT_D):
            d, c = divmod(s, 6)
            pltpu.sync_copy(rot_pack.at[pl.ds(s * _B_PER_W, _B_PER_W)],
                            rot_out.at[d, c, pl.ds(base, _B_PER_W)])
        for s in range(_TRA_D):
            d, c = divmod(s, 3)
            pltpu.sync_copy(tra_pack.at[pl.ds(s * _B_PER_W, _B_PER_W)],
                            tra_out.at[d, c, pl.ds(base, _B_PER_W)])

    return gather_kernel


_GATHER = _make_gather()


def kernel(rotation_per_domain, translation_per_domain, indexes):
    n, d, _ = rotation_per_domain.shape
    rot2 = rotation_per_domain.reshape(n // 2, 2, _ROT_D)
    zr = jnp.zeros((n // 2, _ROT_S - _ROT_D), jnp.float32)
    rot_rows = jnp.concatenate([rot2[:, 0], zr, rot2[:, 1], zr], axis=1)
    tra4 = translation_per_domain.reshape(n // 4, 4, _TRA_D)
    zt = jnp.zeros((n // 4, _TRA_S - _TRA_D), jnp.float32)
    tra_rows = jnp.concatenate(
        [tra4[:, 0], zt, tra4[:, 1], zt, tra4[:, 2], zt, tra4[:, 3], zt],
        axis=1)
    idx = indexes.astype(jnp.int32)
    rot_o, tra_o = _GATHER(rot_rows, tra_rows, idx)
    rot = rot_o[:, :6, :].transpose(2, 0, 1)
    tra = tra_o[:, :3, :].transpose(2, 0, 1)
    return (rot, tra)


# speedup vs baseline: 9.4545x; 9.4545x over previous
import functools

import jax
import jax.numpy as jnp
from jax import lax
from jax.experimental import pallas as pl
from jax.experimental.pallas import tpu as pltpu
from jax.experimental.pallas import tpu_sc as plsc

_N_IMAGES = 150000
_BATCH = 16384
_ROT_D = 36
_TRA_D = 18
_ROT_S = 64
_TRA_S = 32
_L = 16
_CHUNK = 128

_NW = 32
_B_PER_W = _BATCH // _NW
_NCHUNK = _B_PER_W // _CHUNK


def _make_gather():
    mesh = plsc.VectorSubcoreMesh(core_axis_name="c", subcore_axis_name="s")

    @functools.partial(
        pl.kernel,
        mesh=mesh,
        compiler_params=pltpu.CompilerParams(
            use_tc_tiling_on_sc=False, needs_layout_passes=False),
        out_type=[
            jax.ShapeDtypeStruct((6, 8, _BATCH), jnp.float32),
            jax.ShapeDtypeStruct((6, 4, _BATCH), jnp.float32),
        ],
        scratch_types=[
            pltpu.VMEM((_NCHUNK, _CHUNK), jnp.int32),
            pltpu.VMEM((_B_PER_W, _ROT_S), jnp.float32),
            pltpu.VMEM((_B_PER_W, _TRA_S), jnp.float32),
            pltpu.VMEM((_ROT_D * _B_PER_W,), jnp.float32),
            pltpu.VMEM((_TRA_D * _B_PER_W,), jnp.float32),
            pltpu.SemaphoreType.DMA,
        ],
    )
    def gather_kernel(rot_hbm, tra_hbm, idx_hbm, rot_out, tra_out,
                      idx_v, rot_stage, tra_stage, rot_pack, tra_pack, sem):
        wid = lax.axis_index("s") * 2 + lax.axis_index("c")
        base = wid * _B_PER_W
        iota = lax.iota(jnp.int32, _L)

        pltpu.sync_copy(idx_hbm.at[pl.ds(wid * _NCHUNK, _NCHUNK)], idx_v)
        copies = []
        for c in range(_NCHUNK):
            copies.append(pltpu.async_copy(
                rot_hbm.at[idx_v.at[c]],
                rot_stage.at[pl.ds(c * _CHUNK, _CHUNK)], sem))
            copies.append(pltpu.async_copy(
                tra_hbm.at[idx_v.at[c]],
                tra_stage.at[pl.ds(c * _CHUNK, _CHUNK)], sem))
        for cp in copies:
            cp.wait()

        def rot_body(it, _):
            g = it & 31
            s = it >> 5
            j = iota + g * _L
            vals = plsc.load_gather(rot_stage, [j, iota * 0 + s])
            rot_pack[pl.ds(s * _B_PER_W + g * _L, _L)] = vals
            return 0

        lax.fori_loop(0, _ROT_D * (_B_PER_W // _L), rot_body, 0)

        def tra_body(it, _):
            g = it & 31
            s = it >> 5
            j = iota + g * _L
            vals = plsc.load_gather(tra_stage, [j, iota * 0 + s])
            tra_pack[pl.ds(s * _B_PER_W + g * _L, _L)] = vals
            return 0

        lax.fori_loop(0, _TRA_D * (_B_PER_W // _L), tra_body, 0)

        for s in range(_ROT_D):
            d, c = divmod(s, 6)
            pltpu.sync_copy(rot_pack.at[pl.ds(s * _B_PER_W, _B_PER_W)],
                            rot_out.at[d, c, pl.ds(base, _B_PER_W)])
        for s in range(_TRA_D):
            d, c = divmod(s, 3)
            pltpu.sync_copy(tra_pack.at[pl.ds(s * _B_PER_W, _B_PER_W)],
                            tra_out.at[d, c, pl.ds(base, _B_PER_W)])

    return gather_kernel


_GATHER = _make_gather()


def kernel(rotation_per_domain, translation_per_domain, indexes):
    n, d, _ = rotation_per_domain.shape
    rot_rows = jnp.pad(rotation_per_domain.reshape(n, _ROT_D),
                       ((0, 0), (0, _ROT_S - _ROT_D)))
    tra_rows = jnp.pad(translation_per_domain.reshape(n, _TRA_D),
                       ((0, 0), (0, _TRA_S - _TRA_D)))
    idx = indexes.astype(jnp.int32).reshape(_NW * _NCHUNK, _CHUNK)
    rot_o, tra_o = _GATHER(rot_rows, tra_rows, idx)
    rot = rot_o[:, :6, :].transpose(2, 0, 1)
    tra = tra_o[:, :3, :].transpose(2, 0, 1)
    return (rot, tra)
